# baseline (speedup 1.0000x reference)
import functools

import jax
import jax.numpy as jnp
from jax import lax
from jax.experimental import pallas as pl
from jax.experimental.pallas import tpu as pltpu
from jax.experimental.pallas import tpu_sc as plsc

_D0, _D1 = 345, 1024
_N = _D0 * _D1
_NW = 32
_TILE = 256
_CHUNK = 44 * _TILE
_LAST_BASE = _N - _CHUNK

_mesh = plsc.VectorSubcoreMesh(core_axis_name="c", subcore_axis_name="s")


@functools.partial(
    pl.kernel,
    mesh=_mesh,
    out_type=jax.ShapeDtypeStruct((_N,), jnp.float16),
    scratch_types=[pltpu.VMEM((_CHUNK,), jnp.float16)],
)
def _copy_flat(para_hbm, out_hbm, buf):
    wid = lax.axis_index("s") * 2 + lax.axis_index("c")
    base = pl.multiple_of(jnp.minimum(wid * _CHUNK, _LAST_BASE), _TILE)
    pltpu.sync_copy(para_hbm.at[pl.ds(base, _CHUNK)], buf)
    pltpu.sync_copy(buf, out_hbm.at[pl.ds(base, _CHUNK)])


def kernel(para, classes):
    del classes
    return _copy_flat(para.reshape(_N)).reshape(_D0, _D1)


# speedup vs baseline: 1.0337x; 1.0337x over previous
import functools

import jax
import jax.numpy as jnp
from jax import lax
from jax.experimental import pallas as pl
from jax.experimental.pallas import tpu as pltpu
from jax.experimental.pallas import tpu_sc as plsc

_D0, _D1 = 345, 1024
_N = _D0 * _D1
_NC = 1
_NS = 16
_NW = _NC * _NS
_TILE = 256
_NT = _N // _TILE
_CHUNK = -(-_NT // _NW) * _TILE
_LAST_BASE = _N - _CHUNK

_mesh = plsc.VectorSubcoreMesh(
    core_axis_name="c", subcore_axis_name="s", num_cores=_NC, num_subcores=_NS
)


@functools.partial(
    pl.kernel,
    mesh=_mesh,
    out_type=jax.ShapeDtypeStruct((_N,), jnp.float16),
    scratch_types=[pltpu.VMEM((_CHUNK,), jnp.float16)],
)
def _copy_flat(para_hbm, out_hbm, buf):
    wid = lax.axis_index("s") * _NC + lax.axis_index("c")
    base = pl.multiple_of(jnp.minimum(wid * _CHUNK, _LAST_BASE), _TILE)
    pltpu.sync_copy(para_hbm.at[pl.ds(base, _CHUNK)], buf)
    pltpu.sync_copy(buf, out_hbm.at[pl.ds(base, _CHUNK)])


def kernel(para, classes):
    del classes
    return _copy_flat(para.reshape(_N)).reshape(_D0, _D1)
